# Initial kernel scaffold; baseline (speedup 1.0000x reference)
import jax
import jax.numpy as jnp
from jax.experimental import pallas as pl


def kernel(s_parent, mu_p, Sig_p, mask_parent, node_mask, occ_parent):
    raise NotImplementedError("write your pallas kernel here")


# speedup vs baseline: 3.6492x; 3.6492x over previous
import functools

import jax
import jax.numpy as jnp
from jax import lax
from jax.experimental import pallas as pl
from jax.experimental.pallas import tpu as pltpu

OVERSAMPLE_MUL = 6
FPS_KNN = 4
ALPHA = 0.6
SFLOOR = 0.03
SCEIL = 2.0
SIGMA_S = 1.0
JITTER = 1e-06


def _fps_body(mgx_ref, mgy_ref, mgz_ref, l00_ref, l10_ref, l11_ref,
              l20_ref, l21_ref, l22_ref, e0_ref, e1_ref, e2_ref,
              mx_ref, my_ref, mz_ref, d_ref):
    B, M = mgx_ref.shape
    N = mx_ref.shape[1]
    bf = jnp.bfloat16
    f32 = jnp.float32

    def b2(x):
        return x.astype(bf).astype(f32)

    e0 = b2(e0_ref[...])
    e1 = b2(e1_ref[...])
    e2 = b2(e2_ref[...])
    cx = mgx_ref[...] + b2(l00_ref[...]) * e0
    cy = mgy_ref[...] + (b2(l10_ref[...]) * e0 + b2(l11_ref[...]) * e1)
    cz = mgz_ref[...] + ((b2(l20_ref[...]) * e0 + b2(l21_ref[...]) * e1)
                         + b2(l22_ref[...]) * e2)
    lanes = lax.broadcasted_iota(jnp.int32, (B, M), 1)
    nlanes = lax.broadcasted_iota(jnp.int32, (B, N), 1)
    d_ref[...] = jnp.full((B, M), jnp.inf, dtype=jnp.float32)

    def step(t, carry):
        last, mxs, mys, mzs = carry
        onehot = lanes == last
        px = jnp.sum(jnp.where(onehot, cx, 0.0), axis=1, keepdims=True)
        py = jnp.sum(jnp.where(onehot, cy, 0.0), axis=1, keepdims=True)
        pz = jnp.sum(jnp.where(onehot, cz, 0.0), axis=1, keepdims=True)
        sel = nlanes == t
        mxs = jnp.where(sel, px, mxs)
        mys = jnp.where(sel, py, mys)
        mzs = jnp.where(sel, pz, mzs)
        dx = cx - px
        dy = cy - py
        dz = cz - pz
        dist = dx * dx + dy * dy + dz * dz
        d = jnp.minimum(d_ref[...], dist)
        d_ref[...] = d
        vmax = jnp.max(d, axis=1, keepdims=True)
        nxt = jnp.min(jnp.where(d == vmax, lanes, M), axis=1, keepdims=True)
        return nxt, mxs, mys, mzs

    z = jnp.zeros((B, N), dtype=jnp.float32)
    last0 = jnp.zeros((B, 1), dtype=jnp.int32)
    _, mxs, mys, mzs = lax.fori_loop(0, N, step, (last0, z, z, z))
    mx_ref[...] = mxs
    my_ref[...] = mys
    mz_ref[...] = mzs


def _fps_call(mu_g, L_g, eps, N):
    B, M = mu_g.shape[:2]
    out = jax.ShapeDtypeStruct((B, N), jnp.float32)
    ops = (mu_g[..., 0], mu_g[..., 1], mu_g[..., 2],
           L_g[..., 0, 0], L_g[..., 1, 0], L_g[..., 1, 1],
           L_g[..., 2, 0], L_g[..., 2, 1], L_g[..., 2, 2],
           eps[..., 0], eps[..., 1], eps[..., 2])
    return pl.pallas_call(
        _fps_body,
        out_shape=(out, out, out),
        scratch_shapes=[pltpu.VMEM((B, M), jnp.float32)],
    )(*ops)


def _refine_body(m0c_ref, m0r_ref, mupr_ref, logpi_ref, maskp_ref, nmask_ref,
                 s_ref, s0_ref, sig_ref, w_ref):
    N = m0c_ref.shape[0]
    K = logpi_ref.shape[1]
    xc = m0c_ref[:, 0:1]
    yc = m0c_ref[:, 1:2]
    zc = m0c_ref[:, 2:3]
    xr = m0r_ref[0:1, :]
    yr = m0r_ref[1:2, :]
    zr = m0r_ref[2:3, :]
    dxx = xc - xr
    dyy = yc - yr
    dzz = zc - zr
    d2 = dxx * dxx + dyy * dyy + dzz * dzz
    ii = lax.broadcasted_iota(jnp.int32, (N, N), 0)
    jj = lax.broadcasted_iota(jnp.int32, (N, N), 1)
    nmask = nmask_ref[...]
    valid = (nmask > 0.5) & (ii != jj)
    d2m = jnp.where(valid, d2, 1e10)
    acc = jnp.zeros((N, 1), dtype=jnp.float32)
    for _ in range(FPS_KNN):
        m = jnp.min(d2m, axis=1, keepdims=True)
        pos = jnp.min(jnp.where(d2m == m, jj, N), axis=1, keepdims=True)
        d2m = jnp.where(jj == pos, 1e10, d2m)
        acc = acc + jnp.sqrt(jnp.clip(m, 1e-12))
    spacing = acc * (1.0 / FPS_KNN)
    sigma = jnp.clip(ALPHA * spacing, SFLOOR, SCEIL)
    sig_ref[...] = sigma.reshape(1, N)

    mx = mupr_ref[0:1, :]
    my = mupr_ref[1:2, :]
    mz = mupr_ref[2:3, :]
    ax = xc - mx
    ay = yc - my
    az = zc - mz
    dist2p = ax * ax + ay * ay + az * az
    logits = -dist2p / (2.0 * SIGMA_S**2) + logpi_ref[...]
    logits = jnp.where(maskp_ref[...] > 0.5, logits, -1e9)
    lmax = jnp.max(logits, axis=1, keepdims=True)
    e = jnp.exp(logits - lmax)
    w = e / jnp.sum(e, axis=1, keepdims=True)
    w_ref[...] = w
    s0 = jax.lax.dot_general(w, s_ref[...], (((1,), (0,)), ((), ())),
                             preferred_element_type=jnp.float32)
    s0_ref[...] = s0 * jnp.transpose(nmask)


def _refine_call(m0c, m0r, mupr, logpi, maskp, nmask, s_parent):
    B, N, _ = m0c.shape
    K = logpi.shape[2]
    C = s_parent.shape[2]
    grid = (B,)
    bs = lambda shape: pl.BlockSpec((1,) + shape, lambda b: (b,) + (0,) * len(shape))
    out_shapes = (
        jax.ShapeDtypeStruct((B, N, C), jnp.float32),
        jax.ShapeDtypeStruct((B, 1, N), jnp.float32),
        jax.ShapeDtypeStruct((B, N, K), jnp.float32),
    )

    def body(m0c_r, m0r_r, mupr_r, logpi_r, maskp_r, nmask_r, s_r,
             s0_r, sig_r, w_r):
        _refine_body(m0c_r.at[0], m0r_r.at[0], mupr_r.at[0], logpi_r.at[0],
                     maskp_r.at[0], nmask_r.at[0], s_r.at[0],
                     s0_r.at[0], sig_r.at[0], w_r.at[0])

    return pl.pallas_call(
        body,
        grid=grid,
        in_specs=[bs((N, 3)), bs((3, N)), bs((3, K)), bs((1, K)), bs((1, K)),
                  bs((1, N)), bs((K, C))],
        out_specs=(bs((N, C)), bs((1, N)), bs((N, K))),
        out_shape=out_shapes,
    )(m0c, m0r, mupr, logpi, maskp, nmask, s_parent)


def kernel(s_parent, mu_p, Sig_p, mask_parent, node_mask, occ_parent):
    B, K, C = s_parent.shape
    N = node_mask.shape[1]
    M = OVERSAMPLE_MUL * N
    f = s_parent.dtype

    pi = occ_parent * (mask_parent > 0.5).astype(f)
    pi = pi / jnp.clip(jnp.sum(pi, axis=-1, keepdims=True), 1e-09)
    key = jax.random.key(42)
    k1, k2 = jax.random.split(key)
    u = jax.random.uniform(k1, (B, M), dtype=f)
    cdf = jnp.cumsum(pi, axis=-1)
    comp = jnp.clip(jnp.sum((u[:, :, None] > cdf[:, None, :]).astype(jnp.int32),
                            axis=-1), 0, K - 1)
    L = jnp.linalg.cholesky(Sig_p + 1e-06 * jnp.eye(3, dtype=f)[None, None])
    eps = jax.random.normal(k2, (B, M, 3), dtype=f)
    mu_g = jnp.take_along_axis(mu_p, comp[:, :, None], axis=1)
    L_g = jnp.take_along_axis(L, comp[:, :, None, None], axis=1)

    mx, my, mz = _fps_call(mu_g, L_g, eps, N)
    mu0 = jnp.stack([mx, my, mz], axis=-1)
    mu0 = mu0 * node_mask[..., None]

    m0c = mu0
    m0r = jnp.transpose(mu0, (0, 2, 1))
    mupr = jnp.transpose(mu_p, (0, 2, 1))
    logpi = jnp.log(jnp.clip(pi, 1e-09))[:, None, :]
    maskp = mask_parent[:, None, :]
    nmask = node_mask[:, None, :]
    s0, sig, w = _refine_call(m0c, m0r, mupr, logpi, maskp, nmask, s_parent)

    sigma = sig[:, 0, :]
    I3 = jnp.eye(3, dtype=f)[None, None]
    Sig0 = (sigma**2)[..., None, None] * I3
    Sig0 = Sig0 + JITTER * I3 * node_mask[:, :, None, None]
    return s0, mu0, Sig0, w
